# baseline (speedup 1.0000x reference)
import functools

import jax
import jax.numpy as jnp
from jax import lax
from jax.experimental import pallas as pl
from jax.experimental.pallas import tpu as pltpu
from jax.experimental.pallas import tpu_sc as plsc

NC = 2
NS = 16
NW = NC * NS
NBUF = 16
LOOKAHEAD = 8


@jax.jit
def _gather_sc(ids, weight):
    B, F = ids.shape
    D = weight.shape[1]
    rows_per_w = B // NW
    mesh = plsc.VectorSubcoreMesh(core_axis_name="c", subcore_axis_name="s")

    @functools.partial(
        pl.kernel,
        mesh=mesh,
        compiler_params=pltpu.CompilerParams(use_tc_tiling_on_sc=False),
        out_type=jax.ShapeDtypeStruct((B, F, D), jnp.float32),
        scratch_types=[
            pltpu.VMEM((rows_per_w, F), jnp.int32),
            pltpu.VMEM((NBUF, F, D), jnp.float32),
            pltpu.SemaphoreType.DMA,
            pltpu.SemaphoreType.DMA,
        ],
    )
    def k(idx_hbm, table_hbm, out_hbm, idx_v, rows_v, gsem, wsem):
        wid = lax.axis_index("s") * NC + lax.axis_index("c")
        row0 = wid * rows_per_w
        pltpu.sync_copy(idx_hbm.at[pl.ds(row0, rows_per_w)], idx_v)

        def fire_gather(r):
            pltpu.async_copy(
                table_hbm.at[idx_v.at[r]], rows_v.at[r % NBUF], gsem
            )

        def wait_one(sem):
            pltpu.make_async_copy(out_hbm.at[0], rows_v.at[0], sem).wait()

        for r in range(LOOKAHEAD):
            fire_gather(r)

        def body(r, carry):
            @pl.when(r + LOOKAHEAD < rows_per_w)
            def _():
                @pl.when(r >= NBUF - LOOKAHEAD)
                def _():
                    wait_one(wsem)
                fire_gather(r + LOOKAHEAD)

            wait_one(gsem)
            pltpu.async_copy(rows_v.at[r % NBUF], out_hbm.at[row0 + r], wsem)
            return carry

        lax.fori_loop(0, rows_per_w, body, 0)
        for _ in range(NBUF):
            wait_one(wsem)

    return k(ids, weight)


def kernel(token_ids, weight):
    return _gather_sc(token_ids.astype(jnp.int32), weight)
